# baseline (speedup 1.0000x reference)
import jax
import jax.numpy as jnp
from jax import lax
from jax.experimental import pallas as pl
from jax.experimental.pallas import tpu as pltpu, tpu_sc as plsc

_N = 10000
_NPAD = 10240
_G = 64
_D = 256


def _tc1_body(x_ref, brow_ref, w_ref, score_ref, fk_ref, counts_ref, starts_ref):
    x = x_ref[...]
    brow = brow_ref[...]
    w_row = w_ref[...]

    f32 = jnp.float32
    dg = jax.lax.dot_general
    wn = jax.lax.rsqrt(jnp.sum(w_row * w_row))
    score = dg(w_row, x, (((1,), (1,)), ((), ())),
               preferred_element_type=f32) * wn

    sc = jnp.where(score == 0.0, 0.0, score)
    sbits = jax.lax.bitcast_convert_type(sc, jnp.int32)
    key_asc = jnp.where(sbits >= 0, sbits, sbits ^ jnp.int32(0x7FFFFFFF))
    fk = -key_asc

    gids_col = jax.lax.broadcasted_iota(jnp.int32, (_G, 1), 0)
    gids_row = jax.lax.broadcasted_iota(jnp.int32, (1, _G), 1)
    ohT = jnp.where(gids_col == brow, 1.0, 0.0).astype(f32)
    ones_row = jnp.ones((1, _NPAD), f32)
    counts_f = dg(ones_row, ohT, (((1,), (1,)), ((), ())),
                  preferred_element_type=f32)
    slt = jnp.where(gids_col < gids_row, 1.0, 0.0).astype(f32)
    starts_f = dg(counts_f, slt, (((1,), (0,)), ((), ())),
                  preferred_element_type=f32)

    score_ref[...] = score
    fk_ref[...] = fk
    counts_ref[...] = counts_f.astype(jnp.int32)
    starts_ref[...] = starts_f.astype(jnp.int32)


def _splat(v):
    return jnp.full((16,), v, jnp.int32)


def _scal(vec):
    return lax.reduce_max(vec, axes=(0,))


def _sc_body(fk_hbm, counts_hbm, starts_hbm, vtab_hbm, fk_v, counts_v,
             starts_v, row_v, sem):
    wid = lax.axis_index("s") * 2 + lax.axis_index("c")
    pltpu.sync_copy(fk_hbm, fk_v)
    pltpu.sync_copy(counts_hbm, counts_v)
    pltpu.sync_copy(starts_hbm, starts_v)
    lanes = lax.iota(jnp.int32, 16)
    imax = jnp.int32(2147483647)
    imin = jnp.int32(-2147483647 - 1)

    for gi in range(2):
        g = wid * 2 + gi
        gv = _splat(g)
        n = _scal(plsc.load_gather(counts_v, [gv]))
        s = _scal(plsc.load_gather(starts_v, [gv]))
        k = (n + 1) // 2
        nchunk = (n + 15) // 16

        def count_le(t):
            def body(i, acc):
                idx = s + i * 16 + lanes
                valid = (i * 16 + lanes) < n
                v = plsc.load_gather(fk_v, [idx], mask=valid)
                c = jnp.where(valid & (v <= t), 1, 0)
                return acc + lax.reduce_sum(c, axes=(0,))
            return lax.fori_loop(0, nchunk, body, jnp.int32(0))

        def mm_body(i, st):
            lo, hi = st
            idx = s + i * 16 + lanes
            valid = (i * 16 + lanes) < n
            v = plsc.load_gather(fk_v, [idx], mask=valid)
            lo = jnp.minimum(lo, lax.reduce_min(
                jnp.where(valid, v, imax), axes=(0,)))
            hi = jnp.maximum(hi, lax.reduce_max(
                jnp.where(valid, v, imin), axes=(0,)))
            return lo, hi

        lo, hi = lax.fori_loop(0, nchunk, mm_body, (imax, imin))
        lo = jnp.minimum(lo, hi)

        def bs_cond(st):
            a, b = st
            return a < b

        def bs_body(st):
            a, b = st
            mid = (a >> 1) + (b >> 1) + (a & b & 1)
            c = count_le(mid)
            return (jnp.where(c >= k, a, mid + 1),
                    jnp.where(c >= k, mid, b))

        vfk, _ = lax.while_loop(bs_cond, bs_body, (lo, hi))
        nless = count_le(vfk - 1)
        need = k - nless

        def eq_body(i, st):
            cnt, best = st
            idx = s + i * 16 + lanes
            valid = (i * 16 + lanes) < n
            v = plsc.load_gather(fk_v, [idx], mask=valid)
            eq = valid & (v == vfk)
            pref = plsc.cumsum(jnp.where(eq, 1, 0)) + cnt
            hit = eq & (pref == need)
            best = jnp.maximum(best, lax.reduce_max(
                jnp.where(hit, idx, jnp.int32(-1)), axes=(0,)))
            return _scal(pref), best

        _, vidx = lax.fori_loop(0, nchunk, eq_body, (jnp.int32(0),
                                                     jnp.int32(-1)))

        row_v[...] = jnp.where(lanes == 0, vfk,
                               jnp.where(lanes == 1, vidx, 0))
        pltpu.sync_copy(row_v, vtab_hbm.at[g])


def _tc2_body(x_ref, brow_ref, score_ref, fk_ref, vtab_ref, out_ref):
    x = x_ref[...]
    brow = brow_ref[...]
    score = score_ref[...]
    fk = fk_ref[...]
    vtab = vtab_ref[...]

    f32 = jnp.float32
    dg = jax.lax.dot_general
    gids_col = jax.lax.broadcasted_iota(jnp.int32, (_G, 1), 0)
    gids_row = jax.lax.broadcasted_iota(jnp.int32, (1, _G), 1)
    ohT = jnp.where(gids_col == brow, 1.0, 0.0).astype(f32)

    ones_row = jnp.ones((1, _NPAD), f32)
    counts = dg(ones_row, ohT, (((1,), (1,)), ((), ())),
                preferred_element_type=f32)
    k = jnp.ceil(0.5 * counts)

    hi_f = (vtab >> 16).astype(f32)
    lo_f = (vtab & 0xFFFF).astype(f32)
    cols = jax.lax.broadcasted_iota(jnp.int32, (1, 16), 1)
    e0 = jnp.where(cols == 0, 1.0, 0.0).astype(f32)
    e1 = jnp.where(cols == 1, 1.0, 0.0).astype(f32)

    hp = jax.lax.Precision.HIGHEST

    def col_as_row(e, m):
        return dg(e, m, (((1,), (1,)), ((), ())), precision=hp,
                  preferred_element_type=f32)

    def per_node(row):
        return dg(row, ohT, (((1,), (0,)), ((), ())), precision=hp,
                  preferred_element_type=f32)

    vfk_at = (per_node(col_as_row(e0, hi_f)).astype(jnp.int32) * 65536
              + per_node(col_as_row(e0, lo_f)).astype(jnp.int32))
    vidx_at = (per_node(col_as_row(e1, hi_f)).astype(jnp.int32) * 65536
               + per_node(col_as_row(e1, lo_f)).astype(jnp.int32))

    idx = jax.lax.broadcasted_iota(jnp.int32, (1, _NPAD), 1)
    sel = (fk < vfk_at) | ((fk == vfk_at) & (idx <= vidx_at))

    gate = jnp.tanh(score)
    wsel = jnp.where(sel, gate, 0.0)
    ohT_w = ohT * wsel
    pooled = dg(ohT_w, x, (((1,), (0,)), ((), ())),
                preferred_element_type=f32)
    inv = 1.0 / jnp.maximum(k, 1.0)
    eye = jnp.where(gids_col == gids_row, 1.0, 0.0).astype(f32)
    inv_col = dg(eye, inv, (((1,), (1,)), ((), ())),
                 preferred_element_type=f32)
    out_ref[...] = pooled * inv_col


def _make_sc():
    mesh = plsc.VectorSubcoreMesh(core_axis_name="c", subcore_axis_name="s")
    return pl.kernel(
        _sc_body,
        out_type=jax.ShapeDtypeStruct((_G, 16), jnp.int32),
        mesh=mesh,
        compiler_params=pltpu.CompilerParams(needs_layout_passes=False),
        scratch_types=[
            pltpu.VMEM((_NPAD,), jnp.int32),
            pltpu.VMEM((_G,), jnp.int32),
            pltpu.VMEM((_G,), jnp.int32),
            pltpu.VMEM((16,), jnp.int32),
            pltpu.SemaphoreType.DMA,
        ],
    )


def kernel(x, edge_index, batch, w):
    del edge_index
    f32 = jnp.float32
    xp = jnp.zeros((_NPAD, _D), f32).at[:_N].set(x)
    brow = jnp.full((1, _NPAD), _G, jnp.int32).at[0, :_N].set(batch)

    score, fk, counts, starts = pl.pallas_call(
        _tc1_body,
        out_shape=[
            jax.ShapeDtypeStruct((1, _NPAD), f32),
            jax.ShapeDtypeStruct((1, _NPAD), jnp.int32),
            jax.ShapeDtypeStruct((1, _G), jnp.int32),
            jax.ShapeDtypeStruct((1, _G), jnp.int32),
        ],
    )(xp, brow, w.reshape(1, _D))

    vtab = _make_sc()(fk.reshape(_NPAD), counts.reshape(_G),
                      starts.reshape(_G))

    out = pl.pallas_call(
        _tc2_body,
        out_shape=jax.ShapeDtypeStruct((_G, _D), f32),
    )(xp, brow, score, fk, vtab)
    return out
